# Initial kernel scaffold; baseline (speedup 1.0000x reference)
import jax
import jax.numpy as jnp
from jax.experimental import pallas as pl


def kernel(h, edge_index, a):
    raise NotImplementedError("write your pallas kernel here")


# speedup vs baseline: 8.1623x; 8.1623x over previous
import jax
import jax.numpy as jnp
from jax import lax
from jax.experimental import pallas as pl
from jax.experimental.pallas import tpu as pltpu
from jax.experimental.pallas import tpu_sc as plsc

N_NODES = 10000
N_EDGES = 320000
D = 128
NEG_SLOPE = 0.1

NC = 2
NS = 16
L = 16
NW = NC * NS
EPW = N_EDGES // NW
C = 80
NCHUNK = EPW // C
N_PAD = 10240
ROWS_PER_TILE = N_PAD // NS
ZROWS = 80


def _edge_body(h_hbm, src_hbm, dst_hbm, a_hbm,
               num_out, p_out,
               num_acc,
               sidx_v, didx_v, hsrc_v, hdst_v, p_buf, a_v,
               sem0, sem1):
  cid = lax.axis_index("c")
  sid = lax.axis_index("s")
  wid = cid * NS + sid
  lane = lax.iota(jnp.int32, L)

  def zfill(i, _):
    for k in range(D // L):
      hsrc_v[i, pl.ds(k * L, L)] = jnp.zeros((L,), jnp.float32)
    return 0
  lax.fori_loop(0, ZROWS, zfill, 0)
  r0 = sid * ROWS_PER_TILE
  for q in range(ROWS_PER_TILE // ZROWS):
    pltpu.sync_copy(hsrc_v, num_acc.at[pl.ds(r0 + q * ZROWS, ZROWS)])
  plsc.subcore_barrier()

  pltpu.sync_copy(a_hbm, a_v)
  a_regs = [a_v[pl.ds(k * L, L)] for k in range(D // L)]

  def chunk_body(g, _):
    base = wid * EPW + g * C
    pltpu.sync_copy(src_hbm.at[pl.ds(base, C)], sidx_v)
    pltpu.sync_copy(dst_hbm.at[pl.ds(base, C)], didx_v)
    pltpu.async_copy(h_hbm.at[sidx_v], hsrc_v, sem0).wait()
    pltpu.async_copy(h_hbm.at[didx_v], hdst_v, sem1).wait()

    def group_body(j, _):
      ve = jnp.zeros((L,), jnp.float32)
      for ii in range(L):
        i = j * L + ii
        acc = None
        for k in range(D // L):
          hs = hsrc_v[i, pl.ds(k * L, L)]
          hd = hdst_v[i, pl.ds(k * L, L)]
          x = hs + hd
          t = jnp.maximum(x, x * NEG_SLOPE) * a_regs[k]
          acc = t if acc is None else acc + t
        e = acc[0]
        for jj in range(1, L):
          e = e + acc[jj]
        ve = jnp.where(lane == ii, e, ve)
      p_buf[pl.ds(j * L, L)] = jnp.exp(ve)
      return 0
    lax.fori_loop(0, C // L, group_body, 0)

    def w_body(j, _):
      vp = p_buf[pl.ds(j * L, L)]
      for ii in range(L):
        i = j * L + ii
        p = vp[ii]
        for k in range(D // L):
          hsrc_v[i, pl.ds(k * L, L)] = hsrc_v[i, pl.ds(k * L, L)] * p
      return 0
    lax.fori_loop(0, C // L, w_body, 0)

    pltpu.sync_copy(hsrc_v, num_acc.at[didx_v], add=True)
    pltpu.sync_copy(p_buf, p_out.at[pl.ds(base, C)])
    return 0
  lax.fori_loop(0, NCHUNK, chunk_body, 0)

  plsc.subcore_barrier()
  for q in range(ROWS_PER_TILE // ZROWS):
    r = r0 + q * ZROWS
    pltpu.sync_copy(num_acc.at[pl.ds(r, ZROWS)], hsrc_v)
    pltpu.sync_copy(hsrc_v, num_out.at[cid].at[pl.ds(r, ZROWS)])


def _den_body(dst_hbm, p_hbm,
              den_out,
              den_acc,
              didx_v, p_v, prow):
  cid = lax.axis_index("c")
  sid = lax.axis_index("s")
  wid = cid * NS + sid
  lane = lax.iota(jnp.int32, L)

  def zfill(i, _):
    for k in range(D // L):
      prow[i, pl.ds(k * L, L)] = jnp.zeros((L,), jnp.float32)
    return 0
  lax.fori_loop(0, ZROWS, zfill, 0)
  r0 = sid * ROWS_PER_TILE
  for q in range(ROWS_PER_TILE // ZROWS):
    pltpu.sync_copy(prow, den_acc.at[pl.ds(r0 + q * ZROWS, ZROWS)])
  plsc.subcore_barrier()

  def chunk_body(g, _):
    base = wid * EPW + g * C
    pltpu.sync_copy(dst_hbm.at[pl.ds(base, C)], didx_v)
    pltpu.sync_copy(p_hbm.at[pl.ds(base, C)], p_v)

    def w_body(j, _):
      vp = p_v[pl.ds(j * L, L)]
      for ii in range(L):
        i = j * L + ii
        prow[i, pl.ds(0, L)] = jnp.where(lane == 0, vp[ii], 0.0)
      return 0
    lax.fori_loop(0, C // L, w_body, 0)

    pltpu.sync_copy(prow, den_acc.at[didx_v], add=True)
    return 0
  lax.fori_loop(0, NCHUNK, chunk_body, 0)

  plsc.subcore_barrier()
  for q in range(ROWS_PER_TILE // ZROWS):
    r = r0 + q * ZROWS
    pltpu.sync_copy(den_acc.at[pl.ds(r, ZROWS)], prow)
    pltpu.sync_copy(prow, den_out.at[cid].at[pl.ds(r, ZROWS)])


@jax.jit
def _sc_pass(h, src, dst, a):
  mesh = plsc.VectorSubcoreMesh(
      core_axis_name="c", subcore_axis_name="s",
      num_cores=NC, num_subcores=NS)
  num, p = pl.kernel(
      _edge_body,
      out_type=(
          jax.ShapeDtypeStruct((NC, N_PAD, D), jnp.float32),
          jax.ShapeDtypeStruct((N_EDGES,), jnp.float32),
      ),
      mesh=mesh,
      scratch_types=(
          pltpu.VMEM_SHARED((N_PAD, D), jnp.float32),
          pltpu.VMEM((C,), jnp.int32),
          pltpu.VMEM((C,), jnp.int32),
          pltpu.VMEM((C, D), jnp.float32),
          pltpu.VMEM((C, D), jnp.float32),
          pltpu.VMEM((C,), jnp.float32),
          pltpu.VMEM((D,), jnp.float32),
          pltpu.SemaphoreType.DMA,
          pltpu.SemaphoreType.DMA,
      ),
  )(h, src, dst, a)
  den = pl.kernel(
      _den_body,
      out_type=jax.ShapeDtypeStruct((NC, N_PAD, D), jnp.float32),
      mesh=mesh,
      scratch_types=(
          pltpu.VMEM_SHARED((N_PAD, D), jnp.float32),
          pltpu.VMEM((C,), jnp.int32),
          pltpu.VMEM((C,), jnp.float32),
          pltpu.VMEM((C, D), jnp.float32),
      ),
  )(dst, p)
  return num, den


def _combine_body(num_ref, den_ref, out_ref):
  n = num_ref[0] + num_ref[1]
  d = den_ref[0, :, 0] + den_ref[1, :, 0]
  out_ref[...] = n / jnp.maximum(d, 1e-16)[:, None]


@jax.jit
def _combine(num, den):
  R = 1000
  grid = (N_NODES // R,)
  return pl.pallas_call(
      _combine_body,
      grid=grid,
      in_specs=[
          pl.BlockSpec((NC, R, D), lambda i: (0, i, 0)),
          pl.BlockSpec((NC, R, D), lambda i: (0, i, 0)),
      ],
      out_specs=pl.BlockSpec((R, D), lambda i: (i, 0)),
      out_shape=jax.ShapeDtypeStruct((N_NODES, D), jnp.float32),
  )(num, den)


def kernel(h, edge_index, a):
  src = edge_index[0].astype(jnp.int32)
  dst = edge_index[1].astype(jnp.int32)
  num, den = _sc_pass(h, src, dst, a)
  return _combine(num, den)


# speedup vs baseline: 9.5085x; 1.1649x over previous
import jax
import jax.numpy as jnp
from jax import lax
from jax.experimental import pallas as pl
from jax.experimental.pallas import tpu as pltpu
from jax.experimental.pallas import tpu_sc as plsc

N_NODES = 10000
N_EDGES = 320000
D = 128
NEG_SLOPE = 0.1

NC = 2
NS = 16
L = 16
NW = NC * NS
EPW = N_EDGES // NW
C = 80
NCHUNK = EPW // C
N_PAD = 10240
ROWS_PER_TILE = N_PAD // NS
ZROWS = 80
DROWS = N_PAD // L
DROWS_PER_TILE = DROWS // NS


def _edge_body(h_hbm, src_hbm, dst_hbm, a_hbm,
               num_out, den_out,
               num_acc, den_acc,
               sidx_v, didx_v, didx2_v, hsrc_v, hdst_v, p_buf, a_v, prow,
               sem0, sem1):
  cid = lax.axis_index("c")
  sid = lax.axis_index("s")
  wid = cid * NS + sid
  lane = lax.iota(jnp.int32, L)

  def zfill(i, _):
    for k in range(D // L):
      hsrc_v[i, pl.ds(k * L, L)] = jnp.zeros((L,), jnp.float32)
      prow[i, pl.ds(k * L, L)] = jnp.zeros((L,), jnp.float32)
    return 0
  lax.fori_loop(0, ZROWS, zfill, 0)
  r0 = sid * ROWS_PER_TILE
  for q in range(ROWS_PER_TILE // ZROWS):
    pltpu.sync_copy(hsrc_v, num_acc.at[pl.ds(r0 + q * ZROWS, ZROWS)])
  d0 = sid * DROWS_PER_TILE
  pltpu.sync_copy(prow.at[pl.ds(0, DROWS_PER_TILE)],
                  den_acc.at[pl.ds(d0, DROWS_PER_TILE)])
  plsc.subcore_barrier()

  pltpu.sync_copy(a_hbm, a_v)
  a_regs = [a_v[pl.ds(k * L, L)] for k in range(D // L)]

  def chunk_body(g, _):
    base = wid * EPW + g * C
    pltpu.sync_copy(src_hbm.at[pl.ds(base, C)], sidx_v)
    pltpu.sync_copy(dst_hbm.at[pl.ds(base, C)], didx_v)
    pltpu.async_copy(h_hbm.at[sidx_v], hsrc_v, sem0).wait()
    pltpu.async_copy(h_hbm.at[didx_v], hdst_v, sem1).wait()

    def group_body(j, _):
      ve = jnp.zeros((L,), jnp.float32)
      for ii in range(L):
        i = j * L + ii
        acc = None
        for k in range(D // L):
          hs = hsrc_v[i, pl.ds(k * L, L)]
          hd = hdst_v[i, pl.ds(k * L, L)]
          x = hs + hd
          t = jnp.maximum(x, x * NEG_SLOPE) * a_regs[k]
          acc = t if acc is None else acc + t
        e = acc[0]
        for jj in range(1, L):
          e = e + acc[jj]
        ve = jnp.where(lane == ii, e, ve)
      p_buf[pl.ds(j * L, L)] = jnp.exp(ve)
      return 0
    lax.fori_loop(0, C // L, group_body, 0)

    def w_body(j, _):
      vp = p_buf[pl.ds(j * L, L)]
      vd = didx_v[pl.ds(j * L, L)]
      vm = jnp.bitwise_and(vd, 15)
      didx2_v[pl.ds(j * L, L)] = jnp.right_shift(vd, 4)
      for ii in range(L):
        i = j * L + ii
        p = vp[ii]
        for k in range(D // L):
          hsrc_v[i, pl.ds(k * L, L)] = hsrc_v[i, pl.ds(k * L, L)] * p
        prow[i, pl.ds(0, L)] = jnp.where(lane == vm[ii], p, 0.0)
      return 0
    lax.fori_loop(0, C // L, w_body, 0)

    pltpu.sync_copy(hsrc_v, num_acc.at[didx_v], add=True)
    pltpu.sync_copy(prow, den_acc.at[didx2_v], add=True)
    return 0
  lax.fori_loop(0, NCHUNK, chunk_body, 0)

  plsc.subcore_barrier()
  for q in range(ROWS_PER_TILE // ZROWS):
    r = r0 + q * ZROWS
    pltpu.sync_copy(num_acc.at[pl.ds(r, ZROWS)], hsrc_v)
    pltpu.sync_copy(hsrc_v, num_out.at[cid].at[pl.ds(r, ZROWS)])
  pltpu.sync_copy(den_acc.at[pl.ds(d0, DROWS_PER_TILE)],
                  prow.at[pl.ds(0, DROWS_PER_TILE)])
  pltpu.sync_copy(prow.at[pl.ds(0, DROWS_PER_TILE)],
                  den_out.at[cid].at[pl.ds(d0, DROWS_PER_TILE)])


@jax.jit
def _sc_pass(h, src, dst, a):
  mesh = plsc.VectorSubcoreMesh(
      core_axis_name="c", subcore_axis_name="s",
      num_cores=NC, num_subcores=NS)
  num, den = pl.kernel(
      _edge_body,
      out_type=(
          jax.ShapeDtypeStruct((NC, N_PAD, D), jnp.float32),
          jax.ShapeDtypeStruct((NC, DROWS, D), jnp.float32),
      ),
      mesh=mesh,
      scratch_types=(
          pltpu.VMEM_SHARED((N_PAD, D), jnp.float32),
          pltpu.VMEM_SHARED((DROWS, D), jnp.float32),
          pltpu.VMEM((C,), jnp.int32),
          pltpu.VMEM((C,), jnp.int32),
          pltpu.VMEM((C,), jnp.int32),
          pltpu.VMEM((C, D), jnp.float32),
          pltpu.VMEM((C, D), jnp.float32),
          pltpu.VMEM((C,), jnp.float32),
          pltpu.VMEM((D,), jnp.float32),
          pltpu.VMEM((C, D), jnp.float32),
          pltpu.SemaphoreType.DMA,
          pltpu.SemaphoreType.DMA,
      ),
  )(h, src, dst, a)
  return num, den


def _combine_body(num_ref, den_ref, out_ref):
  n = num_ref[0] + num_ref[1]
  d = den_ref[0, :, 0] + den_ref[1, :, 0]
  out_ref[...] = n / jnp.maximum(d, 1e-16)[:, None]


@jax.jit
def _combine(num, den_packed):
  den = den_packed[:, :, :L].reshape(NC, N_PAD, 1)
  R = 1000
  grid = (N_NODES // R,)
  return pl.pallas_call(
      _combine_body,
      grid=grid,
      in_specs=[
          pl.BlockSpec((NC, R, D), lambda i: (0, i, 0)),
          pl.BlockSpec((NC, R, 1), lambda i: (0, i, 0)),
      ],
      out_specs=pl.BlockSpec((R, D), lambda i: (i, 0)),
      out_shape=jax.ShapeDtypeStruct((N_NODES, D), jnp.float32),
  )(num, den)


def kernel(h, edge_index, a):
  src = edge_index[0].astype(jnp.int32)
  dst = edge_index[1].astype(jnp.int32)
  num, den = _sc_pass(h, src, dst, a)
  return _combine(num, den)


# speedup vs baseline: 13.0599x; 1.3735x over previous
import jax
import jax.numpy as jnp
from jax import lax
from jax.experimental import pallas as pl
from jax.experimental.pallas import tpu as pltpu
from jax.experimental.pallas import tpu_sc as plsc

N_NODES = 10000
N_EDGES = 320000
D = 128
NEG_SLOPE = 0.1

NC = 2
NS = 16
L = 16
NW = NC * NS
EPW = N_EDGES // NW
C = 80
NCHUNK = EPW // C
N_PAD = 10240
ROWS_PER_TILE = N_PAD // NS
ZROWS = 80
DROWS = N_PAD // L
DROWS_PER_TILE = DROWS // NS


def _edge_body(h_hbm, src_hbm, dst_hbm, a_hbm,
               num_out, den_out,
               num_acc, den_acc,
               sidx_a, sidx_b, didx_a, didx_b, didx2_v,
               hsrc_a, hsrc_b, hdst_v, p_buf, a_v, prow,
               sem_a, sem_b, sem_d):
  cid = lax.axis_index("c")
  sid = lax.axis_index("s")
  wid = cid * NS + sid
  lane = lax.iota(jnp.int32, L)

  def zfill(i, _):
    for k in range(D // L):
      hsrc_a[i, pl.ds(k * L, L)] = jnp.zeros((L,), jnp.float32)
      prow[i, pl.ds(k * L, L)] = jnp.zeros((L,), jnp.float32)
    return 0
  lax.fori_loop(0, ZROWS, zfill, 0)
  r0 = sid * ROWS_PER_TILE
  for q in range(ROWS_PER_TILE // ZROWS):
    pltpu.sync_copy(hsrc_a, num_acc.at[pl.ds(r0 + q * ZROWS, ZROWS)])
  d0 = sid * DROWS_PER_TILE
  pltpu.sync_copy(prow.at[pl.ds(0, DROWS_PER_TILE)],
                  den_acc.at[pl.ds(d0, DROWS_PER_TILE)])
  plsc.subcore_barrier()

  pltpu.sync_copy(a_hbm, a_v)
  a_regs = [a_v[pl.ds(k * L, L)] for k in range(D // L)]

  def prefetch_src(g, sx, dx, hs, sm):
    base = wid * EPW + g * C
    pltpu.sync_copy(src_hbm.at[pl.ds(base, C)], sx)
    pltpu.sync_copy(dst_hbm.at[pl.ds(base, C)], dx)
    pltpu.async_copy(h_hbm.at[sx], hs, sm)

  def proc(dx, hs, sm, dxn, prefetch_dst):
    pltpu.make_async_copy(h_hbm.at[pl.ds(0, C)], hs, sm).wait()
    pltpu.make_async_copy(h_hbm.at[pl.ds(0, C)], hdst_v, sem_d).wait()

    def group_body(j, _):
      ve = jnp.zeros((L,), jnp.float32)
      for ii in range(L):
        i = j * L + ii
        acc = None
        for k in range(D // L):
          x = hs[i, pl.ds(k * L, L)] + hdst_v[i, pl.ds(k * L, L)]
          t = jnp.maximum(x, x * NEG_SLOPE) * a_regs[k]
          acc = t if acc is None else acc + t
        e = acc[0]
        for jj in range(1, L):
          e = e + acc[jj]
        ve = jnp.where(lane == ii, e, ve)
      p_buf[pl.ds(j * L, L)] = jnp.exp(ve)
      return 0
    lax.fori_loop(0, C // L, group_body, 0)

    def w_body(j, _):
      vp = p_buf[pl.ds(j * L, L)]
      vd = dx[pl.ds(j * L, L)]
      vm = jnp.bitwise_and(vd, 15)
      didx2_v[pl.ds(j * L, L)] = jnp.right_shift(vd, 4)
      for ii in range(L):
        i = j * L + ii
        p = vp[ii]
        for k in range(D // L):
          hs[i, pl.ds(k * L, L)] = hs[i, pl.ds(k * L, L)] * p
        prow[i, pl.ds(0, L)] = jnp.where(lane == vm[ii], p, 0.0)
      return 0
    lax.fori_loop(0, C // L, w_body, 0)

    if prefetch_dst:
      pltpu.async_copy(h_hbm.at[dxn], hdst_v, sem_d)

    pltpu.sync_copy(hs, num_acc.at[dx], add=True)
    pltpu.sync_copy(prow, den_acc.at[didx2_v], add=True)

  prefetch_src(0, sidx_a, didx_a, hsrc_a, sem_a)
  pltpu.async_copy(h_hbm.at[didx_a], hdst_v, sem_d)

  def pair_body(q, _):
    g0 = 2 * q
    prefetch_src(g0 + 1, sidx_b, didx_b, hsrc_b, sem_b)
    proc(didx_a, hsrc_a, sem_a, didx_b, True)
    prefetch_src(g0 + 2, sidx_a, didx_a, hsrc_a, sem_a)
    proc(didx_b, hsrc_b, sem_b, didx_a, True)
    return 0
  lax.fori_loop(0, (NCHUNK - 1) // 2, pair_body, 0)
  proc(didx_a, hsrc_a, sem_a, None, False)

  plsc.subcore_barrier()
  for q in range(ROWS_PER_TILE // ZROWS):
    r = r0 + q * ZROWS
    pltpu.sync_copy(num_acc.at[pl.ds(r, ZROWS)], hsrc_a)
    pltpu.sync_copy(hsrc_a, num_out.at[cid].at[pl.ds(r, ZROWS)])
  pltpu.sync_copy(den_acc.at[pl.ds(d0, DROWS_PER_TILE)],
                  prow.at[pl.ds(0, DROWS_PER_TILE)])
  pltpu.sync_copy(prow.at[pl.ds(0, DROWS_PER_TILE)],
                  den_out.at[cid].at[pl.ds(d0, DROWS_PER_TILE)])


@jax.jit
def _sc_pass(h, src, dst, a):
  mesh = plsc.VectorSubcoreMesh(
      core_axis_name="c", subcore_axis_name="s",
      num_cores=NC, num_subcores=NS)
  num, den = pl.kernel(
      _edge_body,
      out_type=(
          jax.ShapeDtypeStruct((NC, N_PAD, D), jnp.float32),
          jax.ShapeDtypeStruct((NC, DROWS, D), jnp.float32),
      ),
      mesh=mesh,
      scratch_types=(
          pltpu.VMEM_SHARED((N_PAD, D), jnp.float32),
          pltpu.VMEM_SHARED((DROWS, D), jnp.float32),
          pltpu.VMEM((C,), jnp.int32),
          pltpu.VMEM((C,), jnp.int32),
          pltpu.VMEM((C,), jnp.int32),
          pltpu.VMEM((C,), jnp.int32),
          pltpu.VMEM((C,), jnp.int32),
          pltpu.VMEM((C, D), jnp.float32),
          pltpu.VMEM((C, D), jnp.float32),
          pltpu.VMEM((C, D), jnp.float32),
          pltpu.VMEM((C,), jnp.float32),
          pltpu.VMEM((D,), jnp.float32),
          pltpu.VMEM((C, D), jnp.float32),
          pltpu.SemaphoreType.DMA,
          pltpu.SemaphoreType.DMA,
          pltpu.SemaphoreType.DMA,
      ),
  )(h, src, dst, a)
  return num, den


def _combine_body(num_ref, den_ref, out_ref):
  n = num_ref[0] + num_ref[1]
  d = den_ref[0, :, 0] + den_ref[1, :, 0]
  out_ref[...] = n / jnp.maximum(d, 1e-16)[:, None]


@jax.jit
def _combine(num, den_packed):
  den = den_packed[:, :, :L].reshape(NC, N_PAD, 1)
  R = 1000
  grid = (N_NODES // R,)
  return pl.pallas_call(
      _combine_body,
      grid=grid,
      in_specs=[
          pl.BlockSpec((NC, R, D), lambda i: (0, i, 0)),
          pl.BlockSpec((NC, R, 1), lambda i: (0, i, 0)),
      ],
      out_specs=pl.BlockSpec((R, D), lambda i: (i, 0)),
      out_shape=jax.ShapeDtypeStruct((N_NODES, D), jnp.float32),
  )(num, den)


def kernel(h, edge_index, a):
  src = edge_index[0].astype(jnp.int32)
  dst = edge_index[1].astype(jnp.int32)
  num, den = _sc_pass(h, src, dst, a)
  return _combine(num, den)
